# baseline (speedup 1.0000x reference)
import functools

import jax
import jax.numpy as jnp
from jax import lax
from jax.experimental import pallas as pl
from jax.experimental.pallas import tpu as pltpu
from jax.experimental.pallas import tpu_sc as plsc

B, N, K, H, D = 32, 1024, 16, 128, 4
G = B // 2
HH = H // 2
NC, NS = 2, 16
CHUNK = 256
L = 16
PLANE = N * HH
FU = 4


def _sc_agg(uT, w1, adr):
    mesh = plsc.VectorSubcoreMesh(core_axis_name="c", subcore_axis_name="s")

    @functools.partial(
        pl.kernel,
        out_type=jax.ShapeDtypeStruct((2 * G, HH, N), jnp.float32),
        mesh=mesh,
        compiler_params=pltpu.CompilerParams(needs_layout_passes=False),
        scratch_types=[
            pltpu.VMEM((PLANE,), jnp.float32),
            pltpu.VMEM((K, CHUNK), jnp.int32),
            pltpu.VMEM((HH, CHUNK), jnp.float32),
            pltpu.VMEM((HH, CHUNK), jnp.float32),
        ],
    )
    def body(u_hbm, w_hbm, adr_hbm, agg_hbm, w_v, adr_v, u_v, a_v):
        item = lax.axis_index("s") * NC + lax.axis_index("c")
        g = lax.shift_right_logical(item, 1)
        pltpu.sync_copy(w_hbm.at[pl.ds(item * PLANE, PLANE)], w_v)

        def chunk_body(ci, _):
            pltpu.sync_copy(adr_hbm.at[g, :, pl.ds(ci * CHUNK, CHUNK)],
                            adr_v)
            pltpu.sync_copy(u_hbm.at[item, :, pl.ds(ci * CHUNK, CHUNK)],
                            u_v)

            def group_body(gi, _):
                jvs = [adr_v[k, pl.ds(gi * L, L)] for k in range(K)]

                @plsc.parallel_loop(0, HH, 1, unroll=FU)
                def f_body(f):
                    uf = u_v[f, pl.ds(gi * L, L)]
                    acc0 = jnp.zeros((L,), jnp.float32)
                    acc1 = jnp.zeros((L,), jnp.float32)
                    ws = w_v.at[pl.ds(pl.multiple_of(f * N, N), N)]
                    for k in range(0, K, 2):
                        g0 = plsc.load_gather(ws, [jvs[k]])
                        g1 = plsc.load_gather(ws, [jvs[k + 1]])
                        acc0 = acc0 + jnp.maximum(uf + g0, 0.0)
                        acc1 = acc1 + jnp.maximum(uf + g1, 0.0)
                    a_v[f, pl.ds(gi * L, L)] = acc0 + acc1

                return 0

            lax.fori_loop(0, CHUNK // L, group_body, 0)
            pltpu.sync_copy(a_v,
                            agg_hbm.at[item, :, pl.ds(ci * CHUNK, CHUNK)])
            return 0

        lax.fori_loop(0, N // CHUNK, chunk_body, 0)

    return body(uT, w1, adr)


def _ln(x, g, b):
    m = x.mean(-1, keepdims=True)
    v = x.var(-1, keepdims=True)
    return g * (x - m) / jnp.sqrt(v + 1e-5) + b


def _bn(x, g, b):
    m = x.mean(0)
    v = x.var(0)
    return g * (x - m) / jnp.sqrt(v + 1e-5) + b


NT = 512
_CT = lambda a, b: jax.lax.dot_general(a, b, (((0,), (1,)), ((), ())))


def _prep_uw(xb, qT, eWk, ebk):
    uT = jnp.stack([_CT(eWk[:H, h * HH:(h + 1) * HH], xb)
                    - qT[h] + ebk[h * HH:(h + 1) * HH, None]
                    for h in range(2)])
    wT = jnp.stack([_CT(eWk[H:2 * H, h * HH:(h + 1) * HH], xb) + qT[h]
                    for h in range(2)])
    return uT, wT


def _qT(pos_tile, eWk):
    return [_CT(eWk[2 * H:, h * HH:(h + 1) * HH], pos_tile)
            for h in range(2)]


def _tc_embed(node_feat, pos, emb_W, emb_b, emb_g, emb_be, eW0k, eb0k):
    def body(nf_ref, pos_ref, W_ref, b_ref, g_ref, be_ref, eW_ref, eb_ref,
             x_ref, u_ref, w_ref):
        xb = jax.nn.relu(_ln(nf_ref[0] @ W_ref[...] + b_ref[0],
                             g_ref[0], be_ref[0]))
        x_ref[0] = xb
        qT = _qT(pos_ref[0], eW_ref[...])
        uT, wT = _prep_uw(xb, qT, eW_ref[...], eb_ref[0])
        u_ref[0] = uT
        w_ref[0] = wT

    full = lambda s: pl.BlockSpec(s, lambda b, t: (0,) * len(s))
    return pl.pallas_call(
        body,
        grid=(G, N // NT),
        in_specs=[
            pl.BlockSpec((1, NT, D), lambda b, t: (b, t, 0)),
            pl.BlockSpec((1, NT, 2), lambda b, t: (b, t, 0)),
            full((D, H)), full((1, H)), full((1, H)), full((1, H)),
            full((2 * H + 2, H)), full((1, H)),
        ],
        out_specs=[
            pl.BlockSpec((1, NT, H), lambda b, t: (b, t, 0)),
            pl.BlockSpec((1, 2, HH, NT), lambda b, t: (b, 0, 0, t)),
            pl.BlockSpec((1, 2, HH, NT), lambda b, t: (b, 0, 0, t)),
        ],
        out_shape=[
            jax.ShapeDtypeStruct((G, N, H), jnp.float32),
            jax.ShapeDtypeStruct((G, 2, HH, N), jnp.float32),
            jax.ShapeDtypeStruct((G, 2, HH, N), jnp.float32),
        ],
    )(node_feat, pos, emb_W, emb_b[None], emb_g[None], emb_be[None],
      eW0k, eb0k[None])


def _tc_update(x, aggT, mask3, pos, nW, nb, ng, nbe, eWk=None, ebk=None):
    last = eWk is None

    def body(x_ref, a_ref, m_ref, pos_ref, nW_ref, nb_ref, g_ref, be_ref,
             eW_ref, eb_ref, x_out, u_ref=None, w_ref=None):
        nW_all = nW_ref[...]
        z = x_ref[0] @ nW_all[:H]
        for h in range(2):
            z += jax.lax.dot_general(
                a_ref[0, h], nW_all[H + h * HH:H + (h + 1) * HH],
                (((0,), (0,)), ((), ())))
        xb = jax.nn.relu(_ln(z + nb_ref[0], g_ref[0], be_ref[0]))
        xb = xb * m_ref[0, 0][:, None]
        x_out[0] = xb
        if not last:
            qT = _qT(pos_ref[0], eW_ref[...])
            uT, wT = _prep_uw(xb, qT, eW_ref[...], eb_ref[0])
            u_ref[0] = uT
            w_ref[0] = wT

    full = lambda s: pl.BlockSpec(s, lambda b, t: (0,) * len(s))
    out_specs = [pl.BlockSpec((1, NT, H), lambda b, t: (b, t, 0))]
    out_shape = [jax.ShapeDtypeStruct((G, N, H), jnp.float32)]
    if not last:
        out_specs += [pl.BlockSpec((1, 2, HH, NT),
                                   lambda b, t: (b, 0, 0, t))] * 2
        out_shape += [jax.ShapeDtypeStruct((G, 2, HH, N), jnp.float32)] * 2
    if last:
        eWk = jnp.zeros((2 * H + 2, H), jnp.float32)
        ebk = jnp.zeros((H,), jnp.float32)
    return pl.pallas_call(
        body,
        grid=(G, N // NT),
        in_specs=[
            pl.BlockSpec((1, NT, H), lambda b, t: (b, t, 0)),
            pl.BlockSpec((1, 2, HH, NT), lambda b, t: (b, 0, 0, t)),
            pl.BlockSpec((1, 1, NT), lambda b, t: (b, 0, t)),
            pl.BlockSpec((1, NT, 2), lambda b, t: (b, t, 0)),
            full((2 * H, H)), full((1, H)), full((1, H)), full((1, H)),
            full((2 * H + 2, H)), full((1, H)),
        ],
        out_specs=out_specs,
        out_shape=out_shape,
    )(x, aggT, mask3, pos, nW, nb[None], ng[None], nbe[None], eWk, ebk[None])


def _tc_readout(xA, xB, mask3, scalar_feat, p):
    def body(xA_ref, xB_ref, m_ref, s_ref, roW, rob, s1W, s1b, s1g, s1be,
             s2W, s2b, s2g, s2be, h1W, h1b, h1g, h1be, h2W, h2b, out_ref):
        m = m_ref[:, 0, :]
        ms = jnp.clip(m.sum(axis=1, keepdims=True), 1, None)
        x3 = jnp.concatenate([xA_ref[...], xB_ref[...]], axis=0)
        gf = (x3 * m[:, :, None]).sum(axis=1) / ms
        gf = jax.nn.relu(gf @ roW[...] + rob[0])
        s = jax.nn.relu(_bn(s_ref[...] @ s1W[...] + s1b[0], s1g[0], s1be[0]))
        s = jax.nn.relu(_bn(s @ s2W[...] + s2b[0], s2g[0], s2be[0]))
        c = jnp.concatenate([gf, s], axis=-1)
        hh = jax.nn.relu(_bn(c @ h1W[...] + h1b[0], h1g[0], h1be[0]))
        out_ref[...] = hh @ h2W[...] + h2b[0]

    args = [xA, xB, mask3, scalar_feat,
            p['ro_W'], p['ro_b'][None],
            p['s1_W'], p['s1_b'][None], p['s1_g'][None], p['s1_be'][None],
            p['s2_W'], p['s2_b'][None], p['s2_g'][None], p['s2_be'][None],
            p['h1_W'], p['h1_b'][None], p['h1_g'][None], p['h1_be'][None],
            p['h2_W'], p['h2_b'][None]]
    return pl.pallas_call(
        body,
        in_specs=[pl.BlockSpec(a.shape, (lambda nd: lambda: (0,) * nd)(a.ndim))
                  for a in args],
        out_specs=pl.BlockSpec((B, 5), lambda: (0, 0)),
        out_shape=jax.ShapeDtypeStruct((B, 5), jnp.float32),
    )(*args)


def kernel(node_feat, pos, mask, scalar_feat, params, edge_idx):
    p = params
    adr = edge_idx.astype(jnp.int32).transpose(0, 2, 1)
    mask3 = mask[:, None, :]
    eWs = [lp['eW'] * (1.0 / K) for lp in p['mp']]
    ebs = [lp['eb'] * (1.0 / K) for lp in p['mp']]
    nmp = len(p['mp'])

    halves = []
    for s in (slice(0, G), slice(G, B)):
        st = _tc_embed(node_feat[s], pos[s], p['emb_W'], p['emb_b'],
                       p['emb_g'], p['emb_be'], eWs[0], ebs[0])
        halves.append({'x': st[0], 'uT': st[1], 'wT': st[2],
                       'pos': pos[s], 'mask3': mask3[s], 'adr': adr[s]})

    for li, lp in enumerate(p['mp']):
        nxt = (eWs[li + 1], ebs[li + 1]) if li + 1 < nmp else (None, None)
        for hv in halves:
            aggT = _sc_agg(hv['uT'].reshape(2 * G, HH, N),
                           hv['wT'].reshape(-1), hv['adr'])
            outs = _tc_update(hv['x'], aggT.reshape(G, 2, HH, N),
                              hv['mask3'], hv['pos'], lp['nW'], lp['nb'],
                              lp['ng'], lp['nbe'], *nxt)
            if li + 1 < nmp:
                hv['x'], hv['uT'], hv['wT'] = outs
            else:
                (hv['x'],) = outs
    return _tc_readout(halves[0]['x'], halves[1]['x'], mask3, scalar_feat, p)


# speedup vs baseline: 1.0197x; 1.0197x over previous
import functools

import jax
import jax.numpy as jnp
from jax import lax
from jax.experimental import pallas as pl
from jax.experimental.pallas import tpu as pltpu
from jax.experimental.pallas import tpu_sc as plsc

B, N, K, H, D = 32, 1024, 16, 128, 4
G = B // 2
HH = H // 2
NC, NS = 2, 16
CHUNK = 256
L = 16
PLANE = N * HH
FU = 2


def _sc_agg(uT, w1, adr):
    mesh = plsc.VectorSubcoreMesh(core_axis_name="c", subcore_axis_name="s")

    @functools.partial(
        pl.kernel,
        out_type=jax.ShapeDtypeStruct((2 * G, HH, N), jnp.float32),
        mesh=mesh,
        compiler_params=pltpu.CompilerParams(needs_layout_passes=False),
        scratch_types=[
            pltpu.VMEM((PLANE,), jnp.float32),
            pltpu.VMEM((K, CHUNK), jnp.int32),
            pltpu.VMEM((HH, CHUNK), jnp.float32),
            pltpu.VMEM((HH, CHUNK), jnp.float32),
        ],
    )
    def body(u_hbm, w_hbm, adr_hbm, agg_hbm, w_v, adr_v, u_v, a_v):
        item = lax.axis_index("s") * NC + lax.axis_index("c")
        g = lax.shift_right_logical(item, 1)
        pltpu.sync_copy(w_hbm.at[pl.ds(item * PLANE, PLANE)], w_v)

        def chunk_body(ci, _):
            pltpu.sync_copy(adr_hbm.at[g, :, pl.ds(ci * CHUNK, CHUNK)],
                            adr_v)
            pltpu.sync_copy(u_hbm.at[item, :, pl.ds(ci * CHUNK, CHUNK)],
                            u_v)

            @plsc.parallel_loop(0, CHUNK // L, 1, unroll=1)
            def group_body(gi):
                jvs = [adr_v[k, pl.ds(gi * L, L)] for k in range(K)]

                @plsc.parallel_loop(0, HH, 1, unroll=FU)
                def f_body(f):
                    uf = u_v[f, pl.ds(gi * L, L)]
                    acc0 = jnp.zeros((L,), jnp.float32)
                    acc1 = jnp.zeros((L,), jnp.float32)
                    ws = w_v.at[pl.ds(pl.multiple_of(f * N, N), N)]
                    for k in range(0, K, 2):
                        g0 = plsc.load_gather(ws, [jvs[k]])
                        g1 = plsc.load_gather(ws, [jvs[k + 1]])
                        acc0 = acc0 + jnp.maximum(uf + g0, 0.0)
                        acc1 = acc1 + jnp.maximum(uf + g1, 0.0)
                    a_v[f, pl.ds(gi * L, L)] = acc0 + acc1

            pltpu.sync_copy(a_v,
                            agg_hbm.at[item, :, pl.ds(ci * CHUNK, CHUNK)])
            return 0

        lax.fori_loop(0, N // CHUNK, chunk_body, 0)

    return body(uT, w1, adr)


def _ln(x, g, b):
    m = x.mean(-1, keepdims=True)
    v = x.var(-1, keepdims=True)
    return g * (x - m) / jnp.sqrt(v + 1e-5) + b


def _bn(x, g, b):
    m = x.mean(0)
    v = x.var(0)
    return g * (x - m) / jnp.sqrt(v + 1e-5) + b


NT = 512
_CT = lambda a, b: jax.lax.dot_general(a, b, (((0,), (1,)), ((), ())))


def _prep_uw(xb, qT, eWk, ebk):
    uT = jnp.stack([_CT(eWk[:H, h * HH:(h + 1) * HH], xb)
                    - qT[h] + ebk[h * HH:(h + 1) * HH, None]
                    for h in range(2)])
    wT = jnp.stack([_CT(eWk[H:2 * H, h * HH:(h + 1) * HH], xb) + qT[h]
                    for h in range(2)])
    return uT, wT


def _qT(pos_tile, eWk):
    return [_CT(eWk[2 * H:, h * HH:(h + 1) * HH], pos_tile)
            for h in range(2)]


def _tc_embed(node_feat, pos, emb_W, emb_b, emb_g, emb_be, eW0k, eb0k):
    def body(nf_ref, pos_ref, W_ref, b_ref, g_ref, be_ref, eW_ref, eb_ref,
             x_ref, u_ref, w_ref):
        xb = jax.nn.relu(_ln(nf_ref[0] @ W_ref[...] + b_ref[0],
                             g_ref[0], be_ref[0]))
        x_ref[0] = xb
        qT = _qT(pos_ref[0], eW_ref[...])
        uT, wT = _prep_uw(xb, qT, eW_ref[...], eb_ref[0])
        u_ref[0] = uT
        w_ref[0] = wT

    full = lambda s: pl.BlockSpec(s, lambda b, t: (0,) * len(s))
    return pl.pallas_call(
        body,
        grid=(G, N // NT),
        in_specs=[
            pl.BlockSpec((1, NT, D), lambda b, t: (b, t, 0)),
            pl.BlockSpec((1, NT, 2), lambda b, t: (b, t, 0)),
            full((D, H)), full((1, H)), full((1, H)), full((1, H)),
            full((2 * H + 2, H)), full((1, H)),
        ],
        out_specs=[
            pl.BlockSpec((1, NT, H), lambda b, t: (b, t, 0)),
            pl.BlockSpec((1, 2, HH, NT), lambda b, t: (b, 0, 0, t)),
            pl.BlockSpec((1, 2, HH, NT), lambda b, t: (b, 0, 0, t)),
        ],
        out_shape=[
            jax.ShapeDtypeStruct((G, N, H), jnp.float32),
            jax.ShapeDtypeStruct((G, 2, HH, N), jnp.float32),
            jax.ShapeDtypeStruct((G, 2, HH, N), jnp.float32),
        ],
    )(node_feat, pos, emb_W, emb_b[None], emb_g[None], emb_be[None],
      eW0k, eb0k[None])


def _tc_update(x, aggT, mask3, pos, nW, nb, ng, nbe, eWk=None, ebk=None):
    last = eWk is None

    def body(x_ref, a_ref, m_ref, pos_ref, nW_ref, nb_ref, g_ref, be_ref,
             eW_ref, eb_ref, x_out, u_ref=None, w_ref=None):
        nW_all = nW_ref[...]
        z = x_ref[0] @ nW_all[:H]
        for h in range(2):
            z += jax.lax.dot_general(
                a_ref[0, h], nW_all[H + h * HH:H + (h + 1) * HH],
                (((0,), (0,)), ((), ())))
        xb = jax.nn.relu(_ln(z + nb_ref[0], g_ref[0], be_ref[0]))
        xb = xb * m_ref[0, 0][:, None]
        x_out[0] = xb
        if not last:
            qT = _qT(pos_ref[0], eW_ref[...])
            uT, wT = _prep_uw(xb, qT, eW_ref[...], eb_ref[0])
            u_ref[0] = uT
            w_ref[0] = wT

    full = lambda s: pl.BlockSpec(s, lambda b, t: (0,) * len(s))
    out_specs = [pl.BlockSpec((1, NT, H), lambda b, t: (b, t, 0))]
    out_shape = [jax.ShapeDtypeStruct((G, N, H), jnp.float32)]
    if not last:
        out_specs += [pl.BlockSpec((1, 2, HH, NT),
                                   lambda b, t: (b, 0, 0, t))] * 2
        out_shape += [jax.ShapeDtypeStruct((G, 2, HH, N), jnp.float32)] * 2
    if last:
        eWk = jnp.zeros((2 * H + 2, H), jnp.float32)
        ebk = jnp.zeros((H,), jnp.float32)
    return pl.pallas_call(
        body,
        grid=(G, N // NT),
        in_specs=[
            pl.BlockSpec((1, NT, H), lambda b, t: (b, t, 0)),
            pl.BlockSpec((1, 2, HH, NT), lambda b, t: (b, 0, 0, t)),
            pl.BlockSpec((1, 1, NT), lambda b, t: (b, 0, t)),
            pl.BlockSpec((1, NT, 2), lambda b, t: (b, t, 0)),
            full((2 * H, H)), full((1, H)), full((1, H)), full((1, H)),
            full((2 * H + 2, H)), full((1, H)),
        ],
        out_specs=out_specs,
        out_shape=out_shape,
    )(x, aggT, mask3, pos, nW, nb[None], ng[None], nbe[None], eWk, ebk[None])


def _tc_readout(xA, xB, mask3, scalar_feat, p):
    def body(xA_ref, xB_ref, m_ref, s_ref, roW, rob, s1W, s1b, s1g, s1be,
             s2W, s2b, s2g, s2be, h1W, h1b, h1g, h1be, h2W, h2b, out_ref):
        m = m_ref[:, 0, :]
        ms = jnp.clip(m.sum(axis=1, keepdims=True), 1, None)
        x3 = jnp.concatenate([xA_ref[...], xB_ref[...]], axis=0)
        gf = (x3 * m[:, :, None]).sum(axis=1) / ms
        gf = jax.nn.relu(gf @ roW[...] + rob[0])
        s = jax.nn.relu(_bn(s_ref[...] @ s1W[...] + s1b[0], s1g[0], s1be[0]))
        s = jax.nn.relu(_bn(s @ s2W[...] + s2b[0], s2g[0], s2be[0]))
        c = jnp.concatenate([gf, s], axis=-1)
        hh = jax.nn.relu(_bn(c @ h1W[...] + h1b[0], h1g[0], h1be[0]))
        out_ref[...] = hh @ h2W[...] + h2b[0]

    args = [xA, xB, mask3, scalar_feat,
            p['ro_W'], p['ro_b'][None],
            p['s1_W'], p['s1_b'][None], p['s1_g'][None], p['s1_be'][None],
            p['s2_W'], p['s2_b'][None], p['s2_g'][None], p['s2_be'][None],
            p['h1_W'], p['h1_b'][None], p['h1_g'][None], p['h1_be'][None],
            p['h2_W'], p['h2_b'][None]]
    return pl.pallas_call(
        body,
        in_specs=[pl.BlockSpec(a.shape, (lambda nd: lambda: (0,) * nd)(a.ndim))
                  for a in args],
        out_specs=pl.BlockSpec((B, 5), lambda: (0, 0)),
        out_shape=jax.ShapeDtypeStruct((B, 5), jnp.float32),
    )(*args)


def kernel(node_feat, pos, mask, scalar_feat, params, edge_idx):
    p = params
    adr = edge_idx.astype(jnp.int32).transpose(0, 2, 1)
    mask3 = mask[:, None, :]
    eWs = [lp['eW'] * (1.0 / K) for lp in p['mp']]
    ebs = [lp['eb'] * (1.0 / K) for lp in p['mp']]
    nmp = len(p['mp'])

    halves = []
    for s in (slice(0, G), slice(G, B)):
        st = _tc_embed(node_feat[s], pos[s], p['emb_W'], p['emb_b'],
                       p['emb_g'], p['emb_be'], eWs[0], ebs[0])
        halves.append({'x': st[0], 'uT': st[1], 'wT': st[2],
                       'pos': pos[s], 'mask3': mask3[s], 'adr': adr[s]})

    for li, lp in enumerate(p['mp']):
        nxt = (eWs[li + 1], ebs[li + 1]) if li + 1 < nmp else (None, None)
        for hv in halves:
            aggT = _sc_agg(hv['uT'].reshape(2 * G, HH, N),
                           hv['wT'].reshape(-1), hv['adr'])
            outs = _tc_update(hv['x'], aggT.reshape(G, 2, HH, N),
                              hv['mask3'], hv['pos'], lp['nW'], lp['nb'],
                              lp['ng'], lp['nbe'], *nxt)
            if li + 1 < nmp:
                hv['x'], hv['uT'], hv['wT'] = outs
            else:
                (hv['x'],) = outs
    return _tc_readout(halves[0]['x'], halves[1]['x'], mask3, scalar_feat, p)


# speedup vs baseline: 1.0850x; 1.0640x over previous
import functools

import jax
import jax.numpy as jnp
from jax import lax
from jax.experimental import pallas as pl
from jax.experimental.pallas import tpu as pltpu
from jax.experimental.pallas import tpu_sc as plsc

B, N, K, H, D = 32, 1024, 16, 128, 4
G = B // 2
HH = H // 2
NC, NS = 2, 16
CHUNK = 256
L = 16
PLANE = N * HH
FU = 2


def _sc_agg(uT, w1, adr):
    mesh = plsc.VectorSubcoreMesh(core_axis_name="c", subcore_axis_name="s")

    @functools.partial(
        pl.kernel,
        out_type=jax.ShapeDtypeStruct((2 * G, HH, N), jnp.float32),
        mesh=mesh,
        compiler_params=pltpu.CompilerParams(needs_layout_passes=False),
        scratch_types=[
            pltpu.VMEM((PLANE,), jnp.float32),
            pltpu.VMEM((K, CHUNK), jnp.int32),
            pltpu.VMEM((HH, CHUNK), jnp.float32),
            pltpu.VMEM((HH, CHUNK), jnp.float32),
        ],
    )
    def body(u_hbm, w_hbm, adr_hbm, agg_hbm, w_v, adr_v, u_v, a_v):
        item = lax.axis_index("s") * NC + lax.axis_index("c")
        g = lax.shift_right_logical(item, 1)
        pltpu.sync_copy(w_hbm.at[pl.ds(item * PLANE, PLANE)], w_v)

        def chunk_body(ci, _):
            pltpu.sync_copy(adr_hbm.at[g, :, pl.ds(ci * CHUNK, CHUNK)],
                            adr_v)
            pltpu.sync_copy(u_hbm.at[item, :, pl.ds(ci * CHUNK, CHUNK)],
                            u_v)

            @plsc.parallel_loop(0, CHUNK // L, 1, unroll=1)
            def group_body(gi):
                jvs = [adr_v[k, pl.ds(gi * L, L)] for k in range(K)]

                @plsc.parallel_loop(0, HH, 1, unroll=FU)
                def f_body(f):
                    uf = u_v[f, pl.ds(gi * L, L)]
                    acc0 = jnp.zeros((L,), jnp.float32)
                    acc1 = jnp.zeros((L,), jnp.float32)
                    ws = w_v.at[pl.ds(pl.multiple_of(f * N, N), N)]
                    for k in range(0, K, 2):
                        g0 = plsc.load_gather(ws, [jvs[k]])
                        g1 = plsc.load_gather(ws, [jvs[k + 1]])
                        acc0 = acc0 + jnp.maximum(uf + g0, 0.0)
                        acc1 = acc1 + jnp.maximum(uf + g1, 0.0)
                    a_v[f, pl.ds(gi * L, L)] = acc0 + acc1

            pltpu.sync_copy(a_v,
                            agg_hbm.at[item, :, pl.ds(ci * CHUNK, CHUNK)])
            return 0

        lax.fori_loop(0, N // CHUNK, chunk_body, 0)

    return body(uT, w1, adr)


def _ln(x, g, b):
    m = x.mean(-1, keepdims=True)
    v = x.var(-1, keepdims=True)
    return g * (x - m) / jnp.sqrt(v + 1e-5) + b


def _bn(x, g, b):
    m = x.mean(0)
    v = x.var(0)
    return g * (x - m) / jnp.sqrt(v + 1e-5) + b


NT = 1024
_CT = lambda a, b: jax.lax.dot_general(a, b, (((0,), (1,)), ((), ())))


def _prep_uw(xb, qT, eWk, ebk):
    uT = jnp.stack([_CT(eWk[:H, h * HH:(h + 1) * HH], xb)
                    - qT[h] + ebk[h * HH:(h + 1) * HH, None]
                    for h in range(2)])
    wT = jnp.stack([_CT(eWk[H:2 * H, h * HH:(h + 1) * HH], xb) + qT[h]
                    for h in range(2)])
    return uT, wT


def _qT(pos_tile, eWk):
    return [_CT(eWk[2 * H:, h * HH:(h + 1) * HH], pos_tile)
            for h in range(2)]


def _tc_embed(node_feat, pos, emb_W, emb_b, emb_g, emb_be, eW0k, eb0k):
    def body(nf_ref, pos_ref, W_ref, b_ref, g_ref, be_ref, eW_ref, eb_ref,
             x_ref, u_ref, w_ref):
        xb = jax.nn.relu(_ln(nf_ref[0] @ W_ref[...] + b_ref[0],
                             g_ref[0], be_ref[0]))
        x_ref[0] = xb
        qT = _qT(pos_ref[0], eW_ref[...])
        uT, wT = _prep_uw(xb, qT, eW_ref[...], eb_ref[0])
        u_ref[0] = uT
        w_ref[0] = wT

    full = lambda s: pl.BlockSpec(s, lambda b, t: (0,) * len(s))
    return pl.pallas_call(
        body,
        grid=(G, N // NT),
        in_specs=[
            pl.BlockSpec((1, NT, D), lambda b, t: (b, t, 0)),
            pl.BlockSpec((1, NT, 2), lambda b, t: (b, t, 0)),
            full((D, H)), full((1, H)), full((1, H)), full((1, H)),
            full((2 * H + 2, H)), full((1, H)),
        ],
        out_specs=[
            pl.BlockSpec((1, NT, H), lambda b, t: (b, t, 0)),
            pl.BlockSpec((1, 2, HH, NT), lambda b, t: (b, 0, 0, t)),
            pl.BlockSpec((1, 2, HH, NT), lambda b, t: (b, 0, 0, t)),
        ],
        out_shape=[
            jax.ShapeDtypeStruct((G, N, H), jnp.float32),
            jax.ShapeDtypeStruct((G, 2, HH, N), jnp.float32),
            jax.ShapeDtypeStruct((G, 2, HH, N), jnp.float32),
        ],
    )(node_feat, pos, emb_W, emb_b[None], emb_g[None], emb_be[None],
      eW0k, eb0k[None])


def _tc_update(x, aggT, mask3, pos, nW, nb, ng, nbe, eWk=None, ebk=None):
    last = eWk is None

    def body(x_ref, a_ref, m_ref, pos_ref, nW_ref, nb_ref, g_ref, be_ref,
             eW_ref, eb_ref, x_out, u_ref=None, w_ref=None):
        nW_all = nW_ref[...]
        z = x_ref[0] @ nW_all[:H]
        for h in range(2):
            z += jax.lax.dot_general(
                a_ref[0, h], nW_all[H + h * HH:H + (h + 1) * HH],
                (((0,), (0,)), ((), ())))
        xb = jax.nn.relu(_ln(z + nb_ref[0], g_ref[0], be_ref[0]))
        xb = xb * m_ref[0, 0][:, None]
        x_out[0] = xb
        if not last:
            qT = _qT(pos_ref[0], eW_ref[...])
            uT, wT = _prep_uw(xb, qT, eW_ref[...], eb_ref[0])
            u_ref[0] = uT
            w_ref[0] = wT

    full = lambda s: pl.BlockSpec(s, lambda b, t: (0,) * len(s))
    out_specs = [pl.BlockSpec((1, NT, H), lambda b, t: (b, t, 0))]
    out_shape = [jax.ShapeDtypeStruct((G, N, H), jnp.float32)]
    if not last:
        out_specs += [pl.BlockSpec((1, 2, HH, NT),
                                   lambda b, t: (b, 0, 0, t))] * 2
        out_shape += [jax.ShapeDtypeStruct((G, 2, HH, N), jnp.float32)] * 2
    if last:
        eWk = jnp.zeros((2 * H + 2, H), jnp.float32)
        ebk = jnp.zeros((H,), jnp.float32)
    return pl.pallas_call(
        body,
        grid=(G, N // NT),
        in_specs=[
            pl.BlockSpec((1, NT, H), lambda b, t: (b, t, 0)),
            pl.BlockSpec((1, 2, HH, NT), lambda b, t: (b, 0, 0, t)),
            pl.BlockSpec((1, 1, NT), lambda b, t: (b, 0, t)),
            pl.BlockSpec((1, NT, 2), lambda b, t: (b, t, 0)),
            full((2 * H, H)), full((1, H)), full((1, H)), full((1, H)),
            full((2 * H + 2, H)), full((1, H)),
        ],
        out_specs=out_specs,
        out_shape=out_shape,
    )(x, aggT, mask3, pos, nW, nb[None], ng[None], nbe[None], eWk, ebk[None])


def _tc_readout(xA, xB, mask3, scalar_feat, p):
    def body(xA_ref, xB_ref, m_ref, s_ref, roW, rob, s1W, s1b, s1g, s1be,
             s2W, s2b, s2g, s2be, h1W, h1b, h1g, h1be, h2W, h2b, out_ref):
        m = m_ref[:, 0, :]
        ms = jnp.clip(m.sum(axis=1, keepdims=True), 1, None)
        x3 = jnp.concatenate([xA_ref[...], xB_ref[...]], axis=0)
        gf = (x3 * m[:, :, None]).sum(axis=1) / ms
        gf = jax.nn.relu(gf @ roW[...] + rob[0])
        s = jax.nn.relu(_bn(s_ref[...] @ s1W[...] + s1b[0], s1g[0], s1be[0]))
        s = jax.nn.relu(_bn(s @ s2W[...] + s2b[0], s2g[0], s2be[0]))
        c = jnp.concatenate([gf, s], axis=-1)
        hh = jax.nn.relu(_bn(c @ h1W[...] + h1b[0], h1g[0], h1be[0]))
        out_ref[...] = hh @ h2W[...] + h2b[0]

    args = [xA, xB, mask3, scalar_feat,
            p['ro_W'], p['ro_b'][None],
            p['s1_W'], p['s1_b'][None], p['s1_g'][None], p['s1_be'][None],
            p['s2_W'], p['s2_b'][None], p['s2_g'][None], p['s2_be'][None],
            p['h1_W'], p['h1_b'][None], p['h1_g'][None], p['h1_be'][None],
            p['h2_W'], p['h2_b'][None]]
    return pl.pallas_call(
        body,
        in_specs=[pl.BlockSpec(a.shape, (lambda nd: lambda: (0,) * nd)(a.ndim))
                  for a in args],
        out_specs=pl.BlockSpec((B, 5), lambda: (0, 0)),
        out_shape=jax.ShapeDtypeStruct((B, 5), jnp.float32),
    )(*args)


def kernel(node_feat, pos, mask, scalar_feat, params, edge_idx):
    p = params
    adr = edge_idx.astype(jnp.int32).transpose(0, 2, 1)
    mask3 = mask[:, None, :]
    eWs = [lp['eW'] * (1.0 / K) for lp in p['mp']]
    ebs = [lp['eb'] * (1.0 / K) for lp in p['mp']]
    nmp = len(p['mp'])

    halves = []
    for s in (slice(0, G), slice(G, B)):
        st = _tc_embed(node_feat[s], pos[s], p['emb_W'], p['emb_b'],
                       p['emb_g'], p['emb_be'], eWs[0], ebs[0])
        halves.append({'x': st[0], 'uT': st[1], 'wT': st[2],
                       'pos': pos[s], 'mask3': mask3[s], 'adr': adr[s]})

    for li, lp in enumerate(p['mp']):
        nxt = (eWs[li + 1], ebs[li + 1]) if li + 1 < nmp else (None, None)
        for hv in halves:
            aggT = _sc_agg(hv['uT'].reshape(2 * G, HH, N),
                           hv['wT'].reshape(-1), hv['adr'])
            outs = _tc_update(hv['x'], aggT.reshape(G, 2, HH, N),
                              hv['mask3'], hv['pos'], lp['nW'], lp['nb'],
                              lp['ng'], lp['nbe'], *nxt)
            if li + 1 < nmp:
                hv['x'], hv['uT'], hv['wT'] = outs
            else:
                (hv['x'],) = outs
    return _tc_readout(halves[0]['x'], halves[1]['x'], mask3, scalar_feat, p)


# speedup vs baseline: 1.1016x; 1.0153x over previous
import functools

import jax
import jax.numpy as jnp
from jax import lax
from jax.experimental import pallas as pl
from jax.experimental.pallas import tpu as pltpu
from jax.experimental.pallas import tpu_sc as plsc

B, N, K, H, D = 32, 1024, 16, 128, 4
G = B // 2
HH = H // 2
NC, NS = 2, 16
CHUNK = 256
L = 16
PLANE = N * HH
FU = 2


def _sc_agg(uT, w1, adr):
    mesh = plsc.VectorSubcoreMesh(core_axis_name="c", subcore_axis_name="s")

    @functools.partial(
        pl.kernel,
        out_type=jax.ShapeDtypeStruct((2 * G, HH, N), jnp.float32),
        mesh=mesh,
        compiler_params=pltpu.CompilerParams(needs_layout_passes=False),
        scratch_types=[
            pltpu.VMEM((PLANE,), jnp.float32),
            pltpu.VMEM((K, CHUNK), jnp.int32),
            pltpu.VMEM((HH, CHUNK), jnp.float32),
            pltpu.VMEM((HH, CHUNK), jnp.float32),
        ],
    )
    def body(u_hbm, w_hbm, adr_hbm, agg_hbm, w_v, adr_v, u_v, a_v):
        item = lax.axis_index("s") * NC + lax.axis_index("c")
        g = lax.shift_right_logical(item, 1)
        pltpu.sync_copy(w_hbm.at[pl.ds(item * PLANE, PLANE)], w_v)

        def chunk_body(ci, _):
            pltpu.sync_copy(adr_hbm.at[g, :, pl.ds(ci * CHUNK, CHUNK)],
                            adr_v)
            pltpu.sync_copy(u_hbm.at[item, :, pl.ds(ci * CHUNK, CHUNK)],
                            u_v)

            @plsc.parallel_loop(0, CHUNK // L, 1, unroll=1)
            def group_body(gi):
                jvs = [adr_v[k, pl.ds(gi * L, L)] for k in range(K)]

                @plsc.parallel_loop(0, HH, 1, unroll=FU)
                def f_body(f):
                    uf = u_v[f, pl.ds(gi * L, L)]
                    acc0 = jnp.zeros((L,), jnp.float32)
                    acc1 = jnp.zeros((L,), jnp.float32)
                    ws = w_v.at[pl.ds(pl.multiple_of(f * N, N), N)]
                    for k in range(0, K, 2):
                        g0 = plsc.load_gather(ws, [jvs[k]])
                        g1 = plsc.load_gather(ws, [jvs[k + 1]])
                        acc0 = acc0 + jnp.maximum(uf + g0, 0.0)
                        acc1 = acc1 + jnp.maximum(uf + g1, 0.0)
                    a_v[f, pl.ds(gi * L, L)] = acc0 + acc1

            pltpu.sync_copy(a_v,
                            agg_hbm.at[item, :, pl.ds(ci * CHUNK, CHUNK)])
            return 0

        lax.fori_loop(0, N // CHUNK, chunk_body, 0)

    return body(uT, w1, adr)


def _ln(x, g, b):
    m = x.mean(-1, keepdims=True)
    v = x.var(-1, keepdims=True)
    return g * (x - m) / jnp.sqrt(v + 1e-5) + b


def _bn(x, g, b):
    m = x.mean(0)
    v = x.var(0)
    return g * (x - m) / jnp.sqrt(v + 1e-5) + b


NT = 1024
_CT = lambda a, b: jax.lax.dot_general(a, b, (((0,), (1,)), ((), ())))


def _prep_uw(xb, qT, eWk, ebk):
    uT = jnp.stack([_CT(eWk[:H, h * HH:(h + 1) * HH], xb)
                    - qT[h] + ebk[h * HH:(h + 1) * HH, None]
                    for h in range(2)])
    wT = jnp.stack([_CT(eWk[H:2 * H, h * HH:(h + 1) * HH], xb) + qT[h]
                    for h in range(2)])
    return uT, wT


def _qT(pos_tile, eWk):
    return [_CT(eWk[2 * H:, h * HH:(h + 1) * HH], pos_tile)
            for h in range(2)]


def _tc_embed(node_feat, pos, emb_W, emb_b, emb_g, emb_be, eW0k, eb0k):
    def body(nf_ref, pos_ref, W_ref, b_ref, g_ref, be_ref, eW_ref, eb_ref,
             x_ref, u_ref, w_ref):
        xb = jax.nn.relu(_ln(nf_ref[0] @ W_ref[...] + b_ref[0],
                             g_ref[0], be_ref[0]))
        x_ref[0] = xb
        qT = _qT(pos_ref[0], eW_ref[...])
        uT, wT = _prep_uw(xb, qT, eW_ref[...], eb_ref[0])
        u_ref[0] = uT
        w_ref[0] = wT

    full = lambda s: pl.BlockSpec(s, lambda b, t: (0,) * len(s))
    return pl.pallas_call(
        body,
        grid=(G, N // NT),
        in_specs=[
            pl.BlockSpec((1, NT, D), lambda b, t: (b, t, 0)),
            pl.BlockSpec((1, NT, 2), lambda b, t: (b, t, 0)),
            full((D, H)), full((1, H)), full((1, H)), full((1, H)),
            full((2 * H + 2, H)), full((1, H)),
        ],
        out_specs=[
            pl.BlockSpec((1, NT, H), lambda b, t: (b, t, 0)),
            pl.BlockSpec((1, 2, HH, NT), lambda b, t: (b, 0, 0, t)),
            pl.BlockSpec((1, 2, HH, NT), lambda b, t: (b, 0, 0, t)),
        ],
        out_shape=[
            jax.ShapeDtypeStruct((G, N, H), jnp.float32),
            jax.ShapeDtypeStruct((G, 2, HH, N), jnp.float32),
            jax.ShapeDtypeStruct((G, 2, HH, N), jnp.float32),
        ],
    )(node_feat, pos, emb_W, emb_b[None], emb_g[None], emb_be[None],
      eW0k, eb0k[None])


def _tc_update(x, aggT, mask3, pos, nW, nb, ng, nbe, eWk=None, ebk=None):
    last = eWk is None

    def body(x_ref, a_ref, m_ref, pos_ref, nW_ref, nb_ref, g_ref, be_ref,
             eW_ref, eb_ref, x_out, u_ref=None, w_ref=None):
        nW_all = nW_ref[...]
        z = x_ref[0] @ nW_all[:H]
        for h in range(2):
            z += jax.lax.dot_general(
                a_ref[0, h], nW_all[H + h * HH:H + (h + 1) * HH],
                (((0,), (0,)), ((), ())))
        xb = jax.nn.relu(_ln(z + nb_ref[0], g_ref[0], be_ref[0]))
        xb = xb * m_ref[0, 0][:, None]
        if last:
            x_out[...] = xb.sum(axis=0)[None, None]
        else:
            x_out[0] = xb
            qT = _qT(pos_ref[0], eW_ref[...])
            uT, wT = _prep_uw(xb, qT, eW_ref[...], eb_ref[0])
            u_ref[0] = uT
            w_ref[0] = wT

    full = lambda s: pl.BlockSpec(s, lambda b, t: (0,) * len(s))
    if not last:
        out_specs = [pl.BlockSpec((1, NT, H), lambda b, t: (b, t, 0))]
        out_shape = [jax.ShapeDtypeStruct((G, N, H), jnp.float32)]
        out_specs += [pl.BlockSpec((1, 2, HH, NT),
                                   lambda b, t: (b, 0, 0, t))] * 2
        out_shape += [jax.ShapeDtypeStruct((G, 2, HH, N), jnp.float32)] * 2
    else:
        out_specs = [pl.BlockSpec((1, 1, H), lambda b, t: (b, 0, 0))]
        out_shape = [jax.ShapeDtypeStruct((G, 1, H), jnp.float32)]
        eWk = jnp.zeros((2 * H + 2, H), jnp.float32)
        ebk = jnp.zeros((H,), jnp.float32)
    return pl.pallas_call(
        body,
        grid=(G, N // NT),
        in_specs=[
            pl.BlockSpec((1, NT, H), lambda b, t: (b, t, 0)),
            pl.BlockSpec((1, 2, HH, NT), lambda b, t: (b, 0, 0, t)),
            pl.BlockSpec((1, 1, NT), lambda b, t: (b, 0, t)),
            pl.BlockSpec((1, NT, 2), lambda b, t: (b, t, 0)),
            full((2 * H, H)), full((1, H)), full((1, H)), full((1, H)),
            full((2 * H + 2, H)), full((1, H)),
        ],
        out_specs=out_specs,
        out_shape=out_shape,
    )(x, aggT, mask3, pos, nW, nb[None], ng[None], nbe[None], eWk, ebk[None])


def _tc_readout(gfA, gfB, mask3, scalar_feat, p):
    def body(gfA_ref, gfB_ref, m_ref, s_ref, roW, rob, s1W, s1b, s1g, s1be,
             s2W, s2b, s2g, s2be, h1W, h1b, h1g, h1be, h2W, h2b, out_ref):
        m = m_ref[:, 0, :]
        ms = jnp.clip(m.sum(axis=1, keepdims=True), 1, None)
        gf = jnp.concatenate([gfA_ref[:, 0, :], gfB_ref[:, 0, :]],
                             axis=0) / ms
        gf = jax.nn.relu(gf @ roW[...] + rob[0])
        s = jax.nn.relu(_bn(s_ref[...] @ s1W[...] + s1b[0], s1g[0], s1be[0]))
        s = jax.nn.relu(_bn(s @ s2W[...] + s2b[0], s2g[0], s2be[0]))
        c = jnp.concatenate([gf, s], axis=-1)
        hh = jax.nn.relu(_bn(c @ h1W[...] + h1b[0], h1g[0], h1be[0]))
        out_ref[...] = hh @ h2W[...] + h2b[0]

    args = [gfA, gfB, mask3, scalar_feat,
            p['ro_W'], p['ro_b'][None],
            p['s1_W'], p['s1_b'][None], p['s1_g'][None], p['s1_be'][None],
            p['s2_W'], p['s2_b'][None], p['s2_g'][None], p['s2_be'][None],
            p['h1_W'], p['h1_b'][None], p['h1_g'][None], p['h1_be'][None],
            p['h2_W'], p['h2_b'][None]]
    return pl.pallas_call(
        body,
        in_specs=[pl.BlockSpec(a.shape, (lambda nd: lambda: (0,) * nd)(a.ndim))
                  for a in args],
        out_specs=pl.BlockSpec((B, 5), lambda: (0, 0)),
        out_shape=jax.ShapeDtypeStruct((B, 5), jnp.float32),
    )(*args)


def kernel(node_feat, pos, mask, scalar_feat, params, edge_idx):
    p = params
    adr = edge_idx.astype(jnp.int32).transpose(0, 2, 1)
    mask3 = mask[:, None, :]
    eWs = [lp['eW'] * (1.0 / K) for lp in p['mp']]
    ebs = [lp['eb'] * (1.0 / K) for lp in p['mp']]
    nmp = len(p['mp'])

    halves = []
    for s in (slice(0, G), slice(G, B)):
        st = _tc_embed(node_feat[s], pos[s], p['emb_W'], p['emb_b'],
                       p['emb_g'], p['emb_be'], eWs[0], ebs[0])
        halves.append({'x': st[0], 'uT': st[1], 'wT': st[2],
                       'pos': pos[s], 'mask3': mask3[s], 'adr': adr[s]})

    for li, lp in enumerate(p['mp']):
        nxt = (eWs[li + 1], ebs[li + 1]) if li + 1 < nmp else (None, None)
        aggs = [_sc_agg(hv['uT'].reshape(2 * G, HH, N),
                        hv['wT'].reshape(-1), hv['adr']) for hv in halves]
        for hv, aggT in zip(halves, aggs):
            outs = _tc_update(hv['x'], aggT.reshape(G, 2, HH, N),
                              hv['mask3'], hv['pos'], lp['nW'], lp['nb'],
                              lp['ng'], lp['nbe'], *nxt)
            if li + 1 < nmp:
                hv['x'], hv['uT'], hv['wT'] = outs
            else:
                (hv['x'],) = outs
    return _tc_readout(halves[0]['x'], halves[1]['x'], mask3, scalar_feat, p)


# speedup vs baseline: 1.1824x; 1.0734x over previous
import functools

import jax
import jax.numpy as jnp
from jax import lax
from jax.experimental import pallas as pl
from jax.experimental.pallas import tpu as pltpu
from jax.experimental.pallas import tpu_sc as plsc

B, N, K, H, D = 32, 1024, 16, 128, 4
G = B // 2
HH = H // 2
NC, NS = 2, 16
CHUNK = 256
L = 16
PLANE = N * HH
FU = 2


def _sc_agg(uT, w1, adr):
    mesh = plsc.VectorSubcoreMesh(core_axis_name="c", subcore_axis_name="s")

    @functools.partial(
        pl.kernel,
        out_type=jax.ShapeDtypeStruct((2 * G, HH, N), jnp.float32),
        mesh=mesh,
        compiler_params=pltpu.CompilerParams(needs_layout_passes=False),
        scratch_types=[
            pltpu.VMEM((PLANE,), jnp.float32),
            pltpu.VMEM((2, K, CHUNK), jnp.int32),
            pltpu.VMEM((2, HH, CHUNK), jnp.float32),
            pltpu.VMEM((HH, CHUNK), jnp.float32),
            pltpu.SemaphoreType.DMA,
            pltpu.SemaphoreType.DMA,
        ],
    )
    def body(u_hbm, w_hbm, adr_hbm, agg_hbm, w_v, adr2_v, u2_v, a_v,
             sem0, sem1):
        item = lax.axis_index("s") * NC + lax.axis_index("c")
        g = lax.shift_right_logical(item, 1)
        sems = (sem0, sem1)

        def start_in(ci, buf):
            sem = sems[buf]
            pltpu.async_copy(adr_hbm.at[g, :, pl.ds(ci * CHUNK, CHUNK)],
                             adr2_v.at[buf], sem)
            pltpu.async_copy(u_hbm.at[item, :, pl.ds(ci * CHUNK, CHUNK)],
                             u2_v.at[buf], sem)

        def wait_in(ci, buf):
            sem = sems[buf]
            pltpu.make_async_copy(adr_hbm.at[g, :, pl.ds(ci * CHUNK, CHUNK)],
                                  adr2_v.at[buf], sem).wait()
            pltpu.make_async_copy(u_hbm.at[item, :, pl.ds(ci * CHUNK, CHUNK)],
                                  u2_v.at[buf], sem).wait()

        start_in(0, 0)
        pltpu.sync_copy(w_hbm.at[pl.ds(item * PLANE, PLANE)], w_v)

        for ci in range(N // CHUNK):
            buf = ci % 2
            wait_in(ci, buf)
            if ci + 1 < N // CHUNK:
                start_in(ci + 1, 1 - buf)
            adr_v = adr2_v.at[buf]
            u_v = u2_v.at[buf]

            @plsc.parallel_loop(0, CHUNK // L, 1, unroll=1)
            def group_body(gi):
                jvs = [adr_v[k, pl.ds(gi * L, L)] for k in range(K)]

                @plsc.parallel_loop(0, HH, 1, unroll=FU)
                def f_body(f):
                    uf = u_v[f, pl.ds(gi * L, L)]
                    acc0 = jnp.zeros((L,), jnp.float32)
                    acc1 = jnp.zeros((L,), jnp.float32)
                    ws = w_v.at[pl.ds(pl.multiple_of(f * N, N), N)]
                    for k in range(0, K, 2):
                        g0 = plsc.load_gather(ws, [jvs[k]])
                        g1 = plsc.load_gather(ws, [jvs[k + 1]])
                        acc0 = acc0 + jnp.maximum(uf + g0, 0.0)
                        acc1 = acc1 + jnp.maximum(uf + g1, 0.0)
                    a_v[f, pl.ds(gi * L, L)] = acc0 + acc1

            pltpu.sync_copy(a_v,
                            agg_hbm.at[item, :, pl.ds(ci * CHUNK, CHUNK)])

    return body(uT, w1, adr)


def _ln(x, g, b):
    m = x.mean(-1, keepdims=True)
    v = x.var(-1, keepdims=True)
    return g * (x - m) / jnp.sqrt(v + 1e-5) + b


def _bn(x, g, b):
    m = x.mean(0)
    v = x.var(0)
    return g * (x - m) / jnp.sqrt(v + 1e-5) + b


NT = 1024
_CT = lambda a, b: jax.lax.dot_general(a, b, (((0,), (1,)), ((), ())))


def _prep_uw(xb, qT, eWk, ebk):
    uT = jnp.stack([_CT(eWk[:H, h * HH:(h + 1) * HH], xb)
                    - qT[h] + ebk[h * HH:(h + 1) * HH, None]
                    for h in range(2)])
    wT = jnp.stack([_CT(eWk[H:2 * H, h * HH:(h + 1) * HH], xb) + qT[h]
                    for h in range(2)])
    return uT, wT


def _qT(pos_tile, eWk):
    return [_CT(eWk[2 * H:, h * HH:(h + 1) * HH], pos_tile)
            for h in range(2)]


def _tc_embed(node_feat, pos, emb_W, emb_b, emb_g, emb_be, eW0k, eb0k):
    def body(nf_ref, pos_ref, W_ref, b_ref, g_ref, be_ref, eW_ref, eb_ref,
             x_ref, u_ref, w_ref):
        xb = jax.nn.relu(_ln(nf_ref[0] @ W_ref[...] + b_ref[0],
                             g_ref[0], be_ref[0]))
        x_ref[0] = xb
        qT = _qT(pos_ref[0], eW_ref[...])
        uT, wT = _prep_uw(xb, qT, eW_ref[...], eb_ref[0])
        u_ref[0] = uT
        w_ref[0] = wT

    full = lambda s: pl.BlockSpec(s, lambda b, t: (0,) * len(s))
    return pl.pallas_call(
        body,
        grid=(G, N // NT),
        in_specs=[
            pl.BlockSpec((1, NT, D), lambda b, t: (b, t, 0)),
            pl.BlockSpec((1, NT, 2), lambda b, t: (b, t, 0)),
            full((D, H)), full((1, H)), full((1, H)), full((1, H)),
            full((2 * H + 2, H)), full((1, H)),
        ],
        out_specs=[
            pl.BlockSpec((1, NT, H), lambda b, t: (b, t, 0)),
            pl.BlockSpec((1, 2, HH, NT), lambda b, t: (b, 0, 0, t)),
            pl.BlockSpec((1, 2, HH, NT), lambda b, t: (b, 0, 0, t)),
        ],
        out_shape=[
            jax.ShapeDtypeStruct((G, N, H), jnp.float32),
            jax.ShapeDtypeStruct((G, 2, HH, N), jnp.float32),
            jax.ShapeDtypeStruct((G, 2, HH, N), jnp.float32),
        ],
    )(node_feat, pos, emb_W, emb_b[None], emb_g[None], emb_be[None],
      eW0k, eb0k[None])


def _tc_update(x, aggT, mask3, pos, nW, nb, ng, nbe, eWk=None, ebk=None):
    last = eWk is None

    def body(x_ref, a_ref, m_ref, pos_ref, nW_ref, nb_ref, g_ref, be_ref,
             eW_ref, eb_ref, x_out, u_ref=None, w_ref=None):
        nW_all = nW_ref[...]
        z = x_ref[0] @ nW_all[:H]
        for h in range(2):
            z += jax.lax.dot_general(
                a_ref[0, h], nW_all[H + h * HH:H + (h + 1) * HH],
                (((0,), (0,)), ((), ())))
        xb = jax.nn.relu(_ln(z + nb_ref[0], g_ref[0], be_ref[0]))
        xb = xb * m_ref[0, 0][:, None]
        if last:
            x_out[...] = xb.sum(axis=0)[None, None]
        else:
            x_out[0] = xb
            qT = _qT(pos_ref[0], eW_ref[...])
            uT, wT = _prep_uw(xb, qT, eW_ref[...], eb_ref[0])
            u_ref[0] = uT
            w_ref[0] = wT

    full = lambda s: pl.BlockSpec(s, lambda b, t: (0,) * len(s))
    if not last:
        out_specs = [pl.BlockSpec((1, NT, H), lambda b, t: (b, t, 0))]
        out_shape = [jax.ShapeDtypeStruct((G, N, H), jnp.float32)]
        out_specs += [pl.BlockSpec((1, 2, HH, NT),
                                   lambda b, t: (b, 0, 0, t))] * 2
        out_shape += [jax.ShapeDtypeStruct((G, 2, HH, N), jnp.float32)] * 2
    else:
        out_specs = [pl.BlockSpec((1, 1, H), lambda b, t: (b, 0, 0))]
        out_shape = [jax.ShapeDtypeStruct((G, 1, H), jnp.float32)]
        eWk = jnp.zeros((2 * H + 2, H), jnp.float32)
        ebk = jnp.zeros((H,), jnp.float32)
    return pl.pallas_call(
        body,
        grid=(G, N // NT),
        in_specs=[
            pl.BlockSpec((1, NT, H), lambda b, t: (b, t, 0)),
            pl.BlockSpec((1, 2, HH, NT), lambda b, t: (b, 0, 0, t)),
            pl.BlockSpec((1, 1, NT), lambda b, t: (b, 0, t)),
            pl.BlockSpec((1, NT, 2), lambda b, t: (b, t, 0)),
            full((2 * H, H)), full((1, H)), full((1, H)), full((1, H)),
            full((2 * H + 2, H)), full((1, H)),
        ],
        out_specs=out_specs,
        out_shape=out_shape,
    )(x, aggT, mask3, pos, nW, nb[None], ng[None], nbe[None], eWk, ebk[None])


def _tc_readout(gfA, gfB, mask3, scalar_feat, p):
    def body(gfA_ref, gfB_ref, m_ref, s_ref, roW, rob, s1W, s1b, s1g, s1be,
             s2W, s2b, s2g, s2be, h1W, h1b, h1g, h1be, h2W, h2b, out_ref):
        m = m_ref[:, 0, :]
        ms = jnp.clip(m.sum(axis=1, keepdims=True), 1, None)
        gf = jnp.concatenate([gfA_ref[:, 0, :], gfB_ref[:, 0, :]],
                             axis=0) / ms
        gf = jax.nn.relu(gf @ roW[...] + rob[0])
        s = jax.nn.relu(_bn(s_ref[...] @ s1W[...] + s1b[0], s1g[0], s1be[0]))
        s = jax.nn.relu(_bn(s @ s2W[...] + s2b[0], s2g[0], s2be[0]))
        c = jnp.concatenate([gf, s], axis=-1)
        hh = jax.nn.relu(_bn(c @ h1W[...] + h1b[0], h1g[0], h1be[0]))
        out_ref[...] = hh @ h2W[...] + h2b[0]

    args = [gfA, gfB, mask3, scalar_feat,
            p['ro_W'], p['ro_b'][None],
            p['s1_W'], p['s1_b'][None], p['s1_g'][None], p['s1_be'][None],
            p['s2_W'], p['s2_b'][None], p['s2_g'][None], p['s2_be'][None],
            p['h1_W'], p['h1_b'][None], p['h1_g'][None], p['h1_be'][None],
            p['h2_W'], p['h2_b'][None]]
    return pl.pallas_call(
        body,
        in_specs=[pl.BlockSpec(a.shape, (lambda nd: lambda: (0,) * nd)(a.ndim))
                  for a in args],
        out_specs=pl.BlockSpec((B, 5), lambda: (0, 0)),
        out_shape=jax.ShapeDtypeStruct((B, 5), jnp.float32),
    )(*args)


def kernel(node_feat, pos, mask, scalar_feat, params, edge_idx):
    p = params
    adr = edge_idx.astype(jnp.int32).transpose(0, 2, 1)
    mask3 = mask[:, None, :]
    eWs = [lp['eW'] * (1.0 / K) for lp in p['mp']]
    ebs = [lp['eb'] * (1.0 / K) for lp in p['mp']]
    nmp = len(p['mp'])

    halves = []
    for s in (slice(0, G), slice(G, B)):
        st = _tc_embed(node_feat[s], pos[s], p['emb_W'], p['emb_b'],
                       p['emb_g'], p['emb_be'], eWs[0], ebs[0])
        halves.append({'x': st[0], 'uT': st[1], 'wT': st[2],
                       'pos': pos[s], 'mask3': mask3[s], 'adr': adr[s]})

    for li, lp in enumerate(p['mp']):
        nxt = (eWs[li + 1], ebs[li + 1]) if li + 1 < nmp else (None, None)
        aggs = [_sc_agg(hv['uT'].reshape(2 * G, HH, N),
                        hv['wT'].reshape(-1), hv['adr']) for hv in halves]
        for hv, aggT in zip(halves, aggs):
            outs = _tc_update(hv['x'], aggT.reshape(G, 2, HH, N),
                              hv['mask3'], hv['pos'], lp['nW'], lp['nb'],
                              lp['ng'], lp['nbe'], *nxt)
            if li + 1 < nmp:
                hv['x'], hv['uT'], hv['wT'] = outs
            else:
                (hv['x'],) = outs
    return _tc_readout(halves[0]['x'], halves[1]['x'], mask3, scalar_feat, p)
